# baseline (speedup 1.0000x reference)
import functools

import jax
import jax.numpy as jnp
from jax import lax
from jax.experimental import pallas as pl
from jax.experimental.pallas import tpu as pltpu
from jax.experimental.pallas import tpu_sc as plsc

KC = 64
LANES = 16
NCORES = 2
NSUB = 16
NW = NCORES * NSUB
BB = 4
NN = 65536
CPB = NW // BB
CH = NN // CPB
GROUPS = CH // LANES
NSTAT = 5
ACC = KC * LANES
PART = NSTAT * KC


def _stats_body(zs_hbm, xs_hbm, out_hbm,
                zs_v, xs_v, cnt_v, sx_v, sq_v, part_v):
    s = lax.axis_index("s")
    wid = lax.axis_index("c") * NSUB + s
    b = wid // CPB
    start = (wid % CPB) * CH

    pltpu.sync_copy(zs_hbm.at[b, pl.ds(start, CH)], zs_v)
    pltpu.sync_copy(xs_hbm.at[b, pl.ds(2 * start, 2 * CH)], xs_v)

    lane = lax.iota(jnp.int32, LANES)
    lane16 = lane * LANES
    hiota = lane >> 1
    zeros = jnp.zeros((LANES,), jnp.float32)
    ones = jnp.ones((LANES,), jnp.float32)

    @plsc.parallel_loop(0, ACC // LANES, unroll=4)
    def _(i):
        sl = pl.ds(i * LANES, LANES)
        cnt_v[sl] = zeros
        sx_v[sl] = zeros
        sq_v[sl] = zeros

    @plsc.parallel_loop(0, GROUPS, unroll=2)
    def _(i):
        base = i * LANES
        z = zs_v[pl.ds(base, LANES)]
        xa = xs_v[pl.ds(i * 2 * LANES, LANES)]
        xb = xs_v[pl.ds(i * 2 * LANES + LANES, LANES)]
        ze_a = plsc.load_gather(zs_v, [hiota + base])
        ze_b = plsc.load_gather(zs_v, [hiota + (base + 8)])
        slot_c = z * LANES + lane
        slot_a = ze_a * LANES + lane
        slot_b = ze_b * LANES + lane
        plsc.addupdate_scatter(cnt_v, [slot_c], ones)
        plsc.addupdate_scatter(sx_v, [slot_a], xa)
        plsc.addupdate_scatter(sq_v, [slot_a], xa * xa)
        plsc.addupdate_scatter(sx_v, [slot_b], xb)
        plsc.addupdate_scatter(sq_v, [slot_b], xb * xb)

    for ch in range(KC // LANES):
        off = ch * LANES * LANES
        c_sum = plsc.load_gather(cnt_v, [lane16 + off])
        for c in range(1, LANES):
            c_sum = c_sum + plsc.load_gather(cnt_v, [lane16 + (off + c)])
        part_v[pl.ds(0 * KC + ch * LANES, LANES)] = c_sum
        for ref, base_stat in ((sx_v, 1), (sq_v, 3)):
            d0 = plsc.load_gather(ref, [lane16 + off])
            d1 = plsc.load_gather(ref, [lane16 + (off + 1)])
            for c in range(2, LANES, 2):
                d0 = d0 + plsc.load_gather(ref, [lane16 + (off + c)])
                d1 = d1 + plsc.load_gather(ref, [lane16 + (off + c + 1)])
            part_v[pl.ds(base_stat * KC + ch * LANES, LANES)] = d0
            part_v[pl.ds((base_stat + 1) * KC + ch * LANES, LANES)] = d1

    pltpu.sync_copy(part_v, out_hbm.at[wid])


@jax.jit
def _cluster_stats(zs, xsf):
    mesh = plsc.VectorSubcoreMesh(core_axis_name="c", subcore_axis_name="s")
    f = pl.kernel(
        _stats_body,
        mesh=mesh,
        compiler_params=pltpu.CompilerParams(needs_layout_passes=False),
        out_type=jax.ShapeDtypeStruct((NW, PART), jnp.float32),
        scratch_types=[
            pltpu.VMEM((CH,), jnp.int32),
            pltpu.VMEM((2 * CH,), jnp.float32),
            pltpu.VMEM((ACC,), jnp.float32),
            pltpu.VMEM((ACC,), jnp.float32),
            pltpu.VMEM((ACC,), jnp.float32),
            pltpu.VMEM((PART,), jnp.float32),
        ],
    )
    return f(zs, xsf)


def kernel(xs, zs, mu, concentration, rate):
    parts = _cluster_stats(zs.astype(jnp.int32), xs.reshape(BB, 2 * NN))
    st = parts.reshape(BB, CPB, NSTAT, KC).sum(axis=1)
    nks = st[:, 0][..., None]
    sum_x = jnp.stack([st[:, 1], st[:, 2]], axis=-1)
    sum_x2 = jnp.stack([st[:, 3], st[:, 4]], axis=-1)
    eff_samples = nks + 1.0
    hyper_means = (mu[None] + sum_x) / eff_samples
    conc = concentration[None] + nks / 2.0
    rt = rate[None] + 0.5 * (mu[None] ** 2 - eff_samples * hyper_means ** 2 + sum_x2)
    gkey = jax.random.key(42)
    tau = jax.random.gamma(gkey, jnp.broadcast_to(conc, rt.shape)) / rt
    precisions = tau * eff_samples
    nkey = jax.random.key(43)
    mu_sample = hyper_means + jax.random.normal(nkey, hyper_means.shape, dtype=xs.dtype) * jnp.power(precisions, -0.5)
    return jnp.concatenate([hyper_means, precisions, mu_sample], axis=-1)


# speedup vs baseline: 2.7549x; 2.7549x over previous
import functools

import jax
import jax.numpy as jnp
from jax import lax
from jax.experimental import pallas as pl
from jax.experimental.pallas import tpu as pltpu
from jax.experimental.pallas import tpu_sc as plsc

KC = 64
LANES = 16
NCORES = 2
NSUB = 16
NW = NCORES * NSUB
BB = 4
NN = 65536
CPB = NW // BB
CH = NN // CPB
GROUPS = CH // LANES
NSTAT = 5
ACC = LANES * KC


def _stats_body(zs_hbm, x0_hbm, x1_hbm, out_hbm,
                zs_v, x0_v, x1_v, cnt_v, sx0_v, sx1_v, sq0_v, sq1_v, part_v):
    wid = lax.axis_index("c") * NSUB + lax.axis_index("s")
    b = wid // CPB
    start = (wid % CPB) * CH

    pltpu.sync_copy(zs_hbm.at[b, pl.ds(start, CH)], zs_v)
    pltpu.sync_copy(x0_hbm.at[b, pl.ds(start, CH)], x0_v)
    pltpu.sync_copy(x1_hbm.at[b, pl.ds(start, CH)], x1_v)

    lane = lax.iota(jnp.int32, LANES)
    zeros = jnp.zeros((LANES,), jnp.float32)
    ones = jnp.ones((LANES,), jnp.float32)

    def zero_body(i, carry):
        sl = pl.ds(i * LANES, LANES)
        cnt_v[sl] = zeros
        sx0_v[sl] = zeros
        sx1_v[sl] = zeros
        sq0_v[sl] = zeros
        sq1_v[sl] = zeros
        return carry

    lax.fori_loop(0, ACC // LANES, zero_body, 0)

    def body(i, carry):
        sl = pl.ds(i * LANES, LANES)
        z = zs_v[sl]
        x0 = x0_v[sl]
        x1 = x1_v[sl]
        idx = lane * KC + z
        plsc.addupdate_scatter(cnt_v, [idx], ones)
        plsc.addupdate_scatter(sx0_v, [idx], x0)
        plsc.addupdate_scatter(sx1_v, [idx], x1)
        plsc.addupdate_scatter(sq0_v, [idx], x0 * x0)
        plsc.addupdate_scatter(sq1_v, [idx], x1 * x1)
        return carry

    lax.fori_loop(0, GROUPS, body, 0)

    for si, ref in enumerate((cnt_v, sx0_v, sx1_v, sq0_v, sq1_v)):
        for ch in range(KC // LANES):
            acc = ref[pl.ds(ch * LANES, LANES)]
            for r in range(1, LANES):
                acc = acc + ref[pl.ds(r * KC + ch * LANES, LANES)]
            part_v[pl.ds(si * KC + ch * LANES, LANES)] = acc

    pltpu.sync_copy(part_v, out_hbm.at[wid])


@jax.jit
def _cluster_stats(zs, x0, x1):
    mesh = plsc.VectorSubcoreMesh(core_axis_name="c", subcore_axis_name="s")
    f = pl.kernel(
        _stats_body,
        mesh=mesh,
        compiler_params=pltpu.CompilerParams(needs_layout_passes=False),
        out_type=jax.ShapeDtypeStruct((NW, NSTAT * KC), jnp.float32),
        scratch_types=[
            pltpu.VMEM((CH,), jnp.int32),
            pltpu.VMEM((CH,), jnp.float32),
            pltpu.VMEM((CH,), jnp.float32),
            pltpu.VMEM((ACC,), jnp.float32),
            pltpu.VMEM((ACC,), jnp.float32),
            pltpu.VMEM((ACC,), jnp.float32),
            pltpu.VMEM((ACC,), jnp.float32),
            pltpu.VMEM((ACC,), jnp.float32),
            pltpu.VMEM((NSTAT * KC,), jnp.float32),
        ],
    )
    return f(zs, x0, x1)


def kernel(xs, zs, mu, concentration, rate):
    x0 = xs[..., 0]
    x1 = xs[..., 1]
    parts = _cluster_stats(zs.astype(jnp.int32), x0, x1)
    st = parts.reshape(BB, CPB, NSTAT, KC).sum(axis=1)
    nks = st[:, 0][..., None]
    sum_x = jnp.stack([st[:, 1], st[:, 2]], axis=-1)
    sum_x2 = jnp.stack([st[:, 3], st[:, 4]], axis=-1)
    eff_samples = nks + 1.0
    hyper_means = (mu[None] + sum_x) / eff_samples
    conc = concentration[None] + nks / 2.0
    rt = rate[None] + 0.5 * (mu[None] ** 2 - eff_samples * hyper_means ** 2 + sum_x2)
    gkey = jax.random.key(42)
    tau = jax.random.gamma(gkey, jnp.broadcast_to(conc, rt.shape)) / rt
    precisions = tau * eff_samples
    nkey = jax.random.key(43)
    mu_sample = hyper_means + jax.random.normal(nkey, hyper_means.shape, dtype=xs.dtype) * jnp.power(precisions, -0.5)
    return jnp.concatenate([hyper_means, precisions, mu_sample], axis=-1)


# speedup vs baseline: 3.5559x; 1.2908x over previous
import jax
import jax.numpy as jnp
from jax import lax
from jax.experimental import pallas as pl
from jax.experimental.pallas import tpu as pltpu
from jax.experimental.pallas import tpu_sc as plsc

KC = 64
LANES = 16
NCORES = 2
NSUB = 16
NW = NCORES * NSUB
BB = 4
NN = 65536
CPB = NW // BB
CH = NN // CPB
GROUPS = CH // LANES
ACC = KC * LANES


def _reduce_lanes(ref, colbase, off):
    s = plsc.load_gather(ref, [colbase + off])
    for c in range(1, LANES):
        s = s + plsc.load_gather(ref, [colbase + (off + c)])
    return s


def _combine_partials(s, b, part_v, tmp_v, shp, out_hbm, nvec):
    width = nvec * LANES
    pltpu.sync_copy(part_v, shp.at[pl.ds(s * width, width)])
    plsc.subcore_barrier()

    @pl.when(s % CPB == 0)
    def _():
        pltpu.sync_copy(shp.at[pl.ds(s * width, CPB * width)], tmp_v)
        for v in range(nvec):
            acc = tmp_v[pl.ds(v * LANES, LANES)]
            for j in range(1, CPB):
                acc = acc + tmp_v[pl.ds(j * width + v * LANES, LANES)]
            part_v[pl.ds(v * LANES, LANES)] = acc
        pltpu.sync_copy(part_v.at[pl.ds(0, width)], out_hbm.at[b])


def _counts_body(zs_hbm, out_hbm, zs_v, cnt_v, part_v, tmp_v, shp):
    s = lax.axis_index("s")
    wid = lax.axis_index("c") * NSUB + s
    b = wid // CPB
    start = (wid % CPB) * CH

    pltpu.sync_copy(zs_hbm.at[b, pl.ds(start, CH)], zs_v)

    lane = lax.iota(jnp.int32, LANES)
    colbase = lane * LANES
    zeros = jnp.zeros((LANES,), jnp.float32)
    ones = jnp.ones((LANES,), jnp.float32)

    @plsc.parallel_loop(0, ACC // LANES, unroll=4)
    def _(i):
        cnt_v[pl.ds(i * LANES, LANES)] = zeros

    @plsc.parallel_loop(0, GROUPS, unroll=4)
    def _(i):
        z = zs_v[pl.ds(i * LANES, LANES)]
        plsc.addupdate_scatter(cnt_v, [z * LANES + lane], ones)

    for ch in range(KC // LANES):
        part_v[pl.ds(ch * LANES, LANES)] = _reduce_lanes(cnt_v, colbase, ch * ACC // 4)

    _combine_partials(s, b, part_v, tmp_v, shp, out_hbm, KC // LANES)


def _sums_body(zs_hbm, x0_hbm, x1_hbm, out_hbm,
               zs_v, x0_v, x1_v, sx0_v, sx1_v, sq0_v, sq1_v, part_v, tmp_v, shp):
    s = lax.axis_index("s")
    wid = lax.axis_index("c") * NSUB + s
    b = wid // CPB
    start = (wid % CPB) * CH

    pltpu.sync_copy(zs_hbm.at[b, pl.ds(start, CH)], zs_v)
    pltpu.sync_copy(x0_hbm.at[b, pl.ds(start, CH)], x0_v)
    pltpu.sync_copy(x1_hbm.at[b, pl.ds(start, CH)], x1_v)

    lane = lax.iota(jnp.int32, LANES)
    colbase = lane * LANES
    zeros = jnp.zeros((LANES,), jnp.float32)

    @plsc.parallel_loop(0, ACC // LANES, unroll=4)
    def _(i):
        sl = pl.ds(i * LANES, LANES)
        sx0_v[sl] = zeros
        sx1_v[sl] = zeros
        sq0_v[sl] = zeros
        sq1_v[sl] = zeros

    @plsc.parallel_loop(0, GROUPS, unroll=2)
    def _(i):
        sl = pl.ds(i * LANES, LANES)
        z = zs_v[sl]
        x0 = x0_v[sl]
        x1 = x1_v[sl]
        idx = z * LANES + lane
        plsc.addupdate_scatter(sx0_v, [idx], x0)
        plsc.addupdate_scatter(sx1_v, [idx], x1)
        plsc.addupdate_scatter(sq0_v, [idx], x0 * x0)
        plsc.addupdate_scatter(sq1_v, [idx], x1 * x1)

    for si, ref in enumerate((sx0_v, sx1_v, sq0_v, sq1_v)):
        for ch in range(KC // LANES):
            part_v[pl.ds(si * KC + ch * LANES, LANES)] = _reduce_lanes(
                ref, colbase, ch * ACC // 4)

    _combine_partials(s, b, part_v, tmp_v, shp, out_hbm, 4 * KC // LANES)


@jax.jit
def _cluster_stats(zs, x0, x1):
    mesh = plsc.VectorSubcoreMesh(core_axis_name="c", subcore_axis_name="s")
    params = pltpu.CompilerParams(needs_layout_passes=False)
    counts = pl.kernel(
        _counts_body,
        mesh=mesh,
        compiler_params=params,
        out_type=jax.ShapeDtypeStruct((BB, KC), jnp.float32),
        scratch_types=[
            pltpu.VMEM((CH,), jnp.int32),
            pltpu.VMEM((ACC,), jnp.float32),
            pltpu.VMEM((KC,), jnp.float32),
            pltpu.VMEM((CPB * KC,), jnp.float32),
            pltpu.VMEM_SHARED((NSUB * KC,), jnp.float32),
        ],
    )
    sums = pl.kernel(
        _sums_body,
        mesh=mesh,
        compiler_params=params,
        out_type=jax.ShapeDtypeStruct((BB, 4 * KC), jnp.float32),
        scratch_types=[
            pltpu.VMEM((CH,), jnp.int32),
            pltpu.VMEM((CH,), jnp.float32),
            pltpu.VMEM((CH,), jnp.float32),
            pltpu.VMEM((ACC,), jnp.float32),
            pltpu.VMEM((ACC,), jnp.float32),
            pltpu.VMEM((ACC,), jnp.float32),
            pltpu.VMEM((ACC,), jnp.float32),
            pltpu.VMEM((4 * KC,), jnp.float32),
            pltpu.VMEM((CPB * 4 * KC,), jnp.float32),
            pltpu.VMEM_SHARED((NSUB * 4 * KC,), jnp.float32),
        ],
    )
    return counts(zs), sums(zs, x0, x1)


def kernel(xs, zs, mu, concentration, rate):
    x0 = xs[..., 0]
    x1 = xs[..., 1]
    nks_flat, sums = _cluster_stats(zs.astype(jnp.int32), x0, x1)
    nks = nks_flat[..., None]
    st = sums.reshape(BB, 4, KC)
    sum_x = jnp.stack([st[:, 0], st[:, 1]], axis=-1)
    sum_x2 = jnp.stack([st[:, 2], st[:, 3]], axis=-1)
    eff_samples = nks + 1.0
    hyper_means = (mu[None] + sum_x) / eff_samples
    conc = concentration[None] + nks / 2.0
    rt = rate[None] + 0.5 * (mu[None] ** 2 - eff_samples * hyper_means ** 2 + sum_x2)
    gkey = jax.random.key(42)
    tau = jax.random.gamma(gkey, jnp.broadcast_to(conc, rt.shape)) / rt
    precisions = tau * eff_samples
    nkey = jax.random.key(43)
    mu_sample = hyper_means + jax.random.normal(nkey, hyper_means.shape, dtype=xs.dtype) * jnp.power(precisions, -0.5)
    return jnp.concatenate([hyper_means, precisions, mu_sample], axis=-1)


# speedup vs baseline: 3.5583x; 1.0007x over previous
import jax
import jax.numpy as jnp
from jax import lax
from jax.experimental import pallas as pl
from jax.experimental.pallas import tpu as pltpu
from jax.experimental.pallas import tpu_sc as plsc

KC = 64
LANES = 16
NCORES = 2
NSUB = 16
NW = NCORES * NSUB
BB = 4
NN = 65536
CPB = NW // BB
CH = NN // CPB
GROUPS = CH // LANES
ACC = KC * LANES


def _reduce_lanes(ref, colbase, off):
    def body(c, acc):
        return acc + plsc.load_gather(ref, [colbase + (off + c)])
    return lax.fori_loop(1, LANES, body, plsc.load_gather(ref, [colbase + off]))


def _combine_partials(s, b, part_v, tmp_v, shp, out_hbm, nvec):
    width = nvec * LANES
    pltpu.sync_copy(part_v, shp.at[pl.ds(s * width, width)])
    plsc.subcore_barrier()

    @pl.when(s % CPB == 0)
    def _():
        pltpu.sync_copy(shp.at[pl.ds(s * width, CPB * width)], tmp_v)

        def vbody(v, carry):
            def jbody(j, acc):
                return acc + tmp_v[pl.ds(j * width + v * LANES, LANES)]
            part_v[pl.ds(v * LANES, LANES)] = lax.fori_loop(
                1, CPB, jbody, tmp_v[pl.ds(v * LANES, LANES)])
            return carry

        lax.fori_loop(0, nvec, vbody, 0)
        pltpu.sync_copy(part_v.at[pl.ds(0, width)], out_hbm.at[b])


def _counts_body(zs_hbm, out_hbm, zs_v, cnt_v, part_v, tmp_v, shp):
    s = lax.axis_index("s")
    wid = lax.axis_index("c") * NSUB + s
    b = wid // CPB
    start = (wid % CPB) * CH

    pltpu.sync_copy(zs_hbm.at[b, pl.ds(start, CH)], zs_v)

    lane = lax.iota(jnp.int32, LANES)
    colbase = lane * LANES
    zeros = jnp.zeros((LANES,), jnp.float32)
    ones = jnp.ones((LANES,), jnp.float32)

    @plsc.parallel_loop(0, ACC // LANES, unroll=4)
    def _(i):
        cnt_v[pl.ds(i * LANES, LANES)] = zeros

    @plsc.parallel_loop(0, GROUPS, unroll=4)
    def _(i):
        z = zs_v[pl.ds(i * LANES, LANES)]
        plsc.addupdate_scatter(cnt_v, [z * LANES + lane], ones)

    for ch in range(KC // LANES):
        part_v[pl.ds(ch * LANES, LANES)] = _reduce_lanes(cnt_v, colbase, ch * ACC // 4)

    _combine_partials(s, b, part_v, tmp_v, shp, out_hbm, KC // LANES)


def _sums_body(zs_hbm, x0_hbm, x1_hbm, out_hbm,
               zs_v, x0_v, x1_v, sx0_v, sx1_v, sq0_v, sq1_v, part_v, tmp_v, shp):
    s = lax.axis_index("s")
    wid = lax.axis_index("c") * NSUB + s
    b = wid // CPB
    start = (wid % CPB) * CH

    pltpu.sync_copy(zs_hbm.at[b, pl.ds(start, CH)], zs_v)
    pltpu.sync_copy(x0_hbm.at[b, pl.ds(start, CH)], x0_v)
    pltpu.sync_copy(x1_hbm.at[b, pl.ds(start, CH)], x1_v)

    lane = lax.iota(jnp.int32, LANES)
    colbase = lane * LANES
    zeros = jnp.zeros((LANES,), jnp.float32)

    @plsc.parallel_loop(0, ACC // LANES, unroll=4)
    def _(i):
        sl = pl.ds(i * LANES, LANES)
        sx0_v[sl] = zeros
        sx1_v[sl] = zeros
        sq0_v[sl] = zeros
        sq1_v[sl] = zeros

    @plsc.parallel_loop(0, GROUPS, unroll=2)
    def _(i):
        sl = pl.ds(i * LANES, LANES)
        z = zs_v[sl]
        x0 = x0_v[sl]
        x1 = x1_v[sl]
        idx = z * LANES + lane
        plsc.addupdate_scatter(sx0_v, [idx], x0)
        plsc.addupdate_scatter(sx1_v, [idx], x1)
        plsc.addupdate_scatter(sq0_v, [idx], x0 * x0)
        plsc.addupdate_scatter(sq1_v, [idx], x1 * x1)

    for si, ref in enumerate((sx0_v, sx1_v, sq0_v, sq1_v)):
        for ch in range(KC // LANES):
            part_v[pl.ds(si * KC + ch * LANES, LANES)] = _reduce_lanes(
                ref, colbase, ch * ACC // 4)

    _combine_partials(s, b, part_v, tmp_v, shp, out_hbm, 4 * KC // LANES)


@jax.jit
def _cluster_stats(zs, x0, x1):
    mesh = plsc.VectorSubcoreMesh(core_axis_name="c", subcore_axis_name="s")
    params = pltpu.CompilerParams(needs_layout_passes=False)
    counts = pl.kernel(
        _counts_body,
        mesh=mesh,
        compiler_params=params,
        out_type=jax.ShapeDtypeStruct((BB, KC), jnp.float32),
        scratch_types=[
            pltpu.VMEM((CH,), jnp.int32),
            pltpu.VMEM((ACC,), jnp.float32),
            pltpu.VMEM((KC,), jnp.float32),
            pltpu.VMEM((CPB * KC,), jnp.float32),
            pltpu.VMEM_SHARED((NSUB * KC,), jnp.float32),
        ],
    )
    sums = pl.kernel(
        _sums_body,
        mesh=mesh,
        compiler_params=params,
        out_type=jax.ShapeDtypeStruct((BB, 4 * KC), jnp.float32),
        scratch_types=[
            pltpu.VMEM((CH,), jnp.int32),
            pltpu.VMEM((CH,), jnp.float32),
            pltpu.VMEM((CH,), jnp.float32),
            pltpu.VMEM((ACC,), jnp.float32),
            pltpu.VMEM((ACC,), jnp.float32),
            pltpu.VMEM((ACC,), jnp.float32),
            pltpu.VMEM((ACC,), jnp.float32),
            pltpu.VMEM((4 * KC,), jnp.float32),
            pltpu.VMEM((CPB * 4 * KC,), jnp.float32),
            pltpu.VMEM_SHARED((NSUB * 4 * KC,), jnp.float32),
        ],
    )
    return counts(zs), sums(zs, x0, x1)


def kernel(xs, zs, mu, concentration, rate):
    x0 = xs[..., 0]
    x1 = xs[..., 1]
    nks_flat, sums = _cluster_stats(zs.astype(jnp.int32), x0, x1)
    nks = nks_flat[..., None]
    st = sums.reshape(BB, 4, KC)
    sum_x = jnp.stack([st[:, 0], st[:, 1]], axis=-1)
    sum_x2 = jnp.stack([st[:, 2], st[:, 3]], axis=-1)
    eff_samples = nks + 1.0
    hyper_means = (mu[None] + sum_x) / eff_samples
    conc = concentration[None] + nks / 2.0
    rt = rate[None] + 0.5 * (mu[None] ** 2 - eff_samples * hyper_means ** 2 + sum_x2)
    gkey = jax.random.key(42)
    tau = jax.random.gamma(gkey, jnp.broadcast_to(conc, rt.shape)) / rt
    precisions = tau * eff_samples
    nkey = jax.random.key(43)
    mu_sample = hyper_means + jax.random.normal(nkey, hyper_means.shape, dtype=xs.dtype) * jnp.power(precisions, -0.5)
    return jnp.concatenate([hyper_means, precisions, mu_sample], axis=-1)
